# baseline (speedup 1.0000x reference)
import dataclasses

import jax
import jax.numpy as jnp
from jax import lax
from jax.experimental import pallas as pl
from jax.experimental.pallas import tpu as pltpu
from jax.experimental.pallas import tpu_sc as plsc

_LANES = 16
_TILES = 32
_SC_BATCHES = 2


def _stats_core(x):
    l, d = x.shape
    xb = x.astype(jnp.bfloat16)
    ones = jnp.ones((8, l), jnp.bfloat16)
    px8 = jax.lax.dot(ones, xb, preferred_element_type=jnp.float32)
    px = px8[0:1]
    rng = jax.lax.broadcasted_iota(jnp.int32, (1, d), 1).astype(jnp.float32) - d / 2.0
    px = px / jnp.sum(px)
    mean = jnp.sum(px * rng)
    std = jnp.sqrt(jnp.sum(px * (rng - mean) ** 2))
    new_idx = (rng - mean) / std + d / 2.0
    ii = new_idx.astype(jnp.int32)
    fl = jnp.clip(ii, 0, d - 1)
    ce = jnp.clip(ii + 1, 0, d - 1)
    w = new_idx - jnp.floor(new_idx)
    return xb, fl, ce, w


def _fused_body(x_ref, o_ref):
    x = x_ref[0]
    l, d = x.shape
    xb, fl, ce, w = _stats_core(x)
    rows = jax.lax.broadcasted_iota(jnp.int32, (d, d), 0)
    g = jnp.where(rows == fl, 1.0 - w, 0.0) + jnp.where(rows == ce, w, 0.0)
    o_ref[0] = jax.lax.dot(
        xb, g.astype(jnp.bfloat16), preferred_element_type=jnp.float32
    )


def _stats_body(x_ref, fl_ref, ce_ref, w_ref):
    _, fl, ce, w = _stats_core(x_ref[0])
    fl_ref[0] = fl
    ce_ref[0] = ce
    w_ref[0] = w


def _make_sc_gather_body(batch_off):
    def _sc_gather_body(x_hbm, fl_hbm, ce_hbm, w_hbm, o_hbm, rows_in, rows_out,
                        flv, cev, wv):
        nb, l, d = o_hbm.shape
        rows_per_tile = l // _TILES
        wid = lax.axis_index("s") * 2 + lax.axis_index("c")
        rbase = wid * rows_per_tile

        @pl.loop(0, nb)
        def _batch(bi):
            pltpu.sync_copy(fl_hbm.at[bi], flv)
            pltpu.sync_copy(ce_hbm.at[bi], cev)
            pltpu.sync_copy(w_hbm.at[bi], wv)
            pltpu.sync_copy(
                x_hbm.at[bi + batch_off, pl.ds(rbase, rows_per_tile)], rows_in
            )

            @pl.loop(0, d, step=_LANES)
            def _grp(c):
                f_idx = flv[pl.ds(c, _LANES)]
                c_idx = cev[pl.ds(c, _LANES)]
                wvec = wv[pl.ds(c, _LANES)]

                @pl.loop(0, rows_per_tile)
                def _row(r):
                    rvec = jnp.full((_LANES,), 0, jnp.int32) + r
                    gf = plsc.load_gather(rows_in, [rvec, f_idx])
                    gc = plsc.load_gather(rows_in, [rvec, c_idx])
                    rows_out[r, pl.ds(c, _LANES)] = gf + wvec * (gc - gf)

            pltpu.sync_copy(rows_out, o_hbm.at[bi, pl.ds(rbase, rows_per_tile)])

    return _sc_gather_body


def kernel(distance):
    b, l, d = distance.shape
    k = _SC_BATCHES
    bt = b - k
    i32 = jnp.int32

    tc_out = pl.pallas_call(
        _fused_body,
        grid=(bt,),
        in_specs=[pl.BlockSpec((1, l, d), lambda i: (i, 0, 0))],
        out_specs=pl.BlockSpec((1, l, d), lambda i: (i, 0, 0)),
        out_shape=jax.ShapeDtypeStruct((bt, l, d), distance.dtype),
    )(distance)

    fl, ce, w = pl.pallas_call(
        _stats_body,
        grid=(k,),
        in_specs=[pl.BlockSpec((1, l, d), lambda i: (i + bt, 0, 0))],
        out_specs=[
            pl.BlockSpec((1, 1, d), lambda i: (i, 0, 0)),
            pl.BlockSpec((1, 1, d), lambda i: (i, 0, 0)),
            pl.BlockSpec((1, 1, d), lambda i: (i, 0, 0)),
        ],
        out_shape=[
            jax.ShapeDtypeStruct((k, 1, d), i32),
            jax.ShapeDtypeStruct((k, 1, d), i32),
            jax.ShapeDtypeStruct((k, 1, d), jnp.float32),
        ],
    )(distance)
    fl, ce, w = fl.reshape(k, d), ce.reshape(k, d), w.reshape(k, d)

    rows_per_tile = l // _TILES
    mesh = plsc.VectorSubcoreMesh(core_axis_name="c", subcore_axis_name="s")
    cp = pltpu.CompilerParams()
    if "needs_layout_passes" in pltpu.CompilerParams.__dataclass_fields__:
        cp = dataclasses.replace(cp, needs_layout_passes=False)
    sc_out = pl.kernel(
        _make_sc_gather_body(bt),
        out_type=jax.ShapeDtypeStruct((k, l, d), jnp.float32),
        mesh=mesh,
        scratch_types=[
            pltpu.VMEM((rows_per_tile, d), jnp.float32),
            pltpu.VMEM((rows_per_tile, d), jnp.float32),
            pltpu.VMEM((d,), i32),
            pltpu.VMEM((d,), i32),
            pltpu.VMEM((d,), jnp.float32),
        ],
        compiler_params=cp,
    )(distance, fl, ce, w)

    return jnp.concatenate([tc_out, sc_out], axis=0)


# speedup vs baseline: 1.0010x; 1.0010x over previous
import dataclasses

import jax
import jax.numpy as jnp
from jax import lax
from jax.experimental import pallas as pl
from jax.experimental.pallas import tpu as pltpu
from jax.experimental.pallas import tpu_sc as plsc

_LANES = 16
_TILES = 32
_SC_BATCHES = 2


def _stats_core(x):
    l, d = x.shape
    xb = x.astype(jnp.bfloat16)
    ones = jnp.ones((8, l), jnp.bfloat16)
    px8 = jax.lax.dot(ones, xb, preferred_element_type=jnp.float32)
    px = px8[0:1]
    rng = jax.lax.broadcasted_iota(jnp.int32, (1, d), 1).astype(jnp.float32) - d / 2.0
    px = px / jnp.sum(px)
    mean = jnp.sum(px * rng)
    std = jnp.sqrt(jnp.sum(px * (rng - mean) ** 2))
    new_idx = (rng - mean) / std + d / 2.0
    ii = new_idx.astype(jnp.int32)
    fl = jnp.clip(ii, 0, d - 1)
    ce = jnp.clip(ii + 1, 0, d - 1)
    w = new_idx - jnp.floor(new_idx)
    return xb, fl, ce, w


def _fused_body(x_ref, o_ref):
    x = x_ref[0]
    l, d = x.shape
    xb, fl, ce, w = _stats_core(x)
    rows = jax.lax.broadcasted_iota(jnp.int32, (d, d), 0)
    g = jnp.where(rows == fl, 1.0 - w, 0.0) + jnp.where(rows == ce, w, 0.0)
    o_ref[0] = jax.lax.dot(
        xb, g.astype(jnp.bfloat16), preferred_element_type=jnp.float32
    )


def _stats_body(x_ref, fl_ref, ce_ref, w_ref):
    _, fl, ce, w = _stats_core(x_ref[0])
    fl_ref[0] = fl
    ce_ref[0] = ce
    w_ref[0] = w


def _make_sc_gather_body(batch_off):
    def _sc_gather_body(x_hbm, fl_hbm, ce_hbm, w_hbm, o_hbm, rows_in, rows_out,
                        flv, cev, wv):
        nb, l, d = o_hbm.shape
        rows_per_tile = l // _TILES
        wid = lax.axis_index("s") * 2 + lax.axis_index("c")
        rbase = wid * rows_per_tile

        @pl.loop(0, nb)
        def _batch(bi):
            pltpu.sync_copy(fl_hbm.at[bi], flv)
            pltpu.sync_copy(ce_hbm.at[bi], cev)
            pltpu.sync_copy(w_hbm.at[bi], wv)
            pltpu.sync_copy(
                x_hbm.at[bi + batch_off, pl.ds(rbase, rows_per_tile)], rows_in
            )

            @pl.loop(0, d, step=_LANES)
            def _grp(c):
                f_idx = flv[pl.ds(c, _LANES)]
                c_idx = cev[pl.ds(c, _LANES)]
                wvec = wv[pl.ds(c, _LANES)]

                @pl.loop(0, rows_per_tile)
                def _row(r):
                    rvec = jnp.full((_LANES,), 0, jnp.int32) + r
                    gf = plsc.load_gather(rows_in, [rvec, f_idx])
                    gc = plsc.load_gather(rows_in, [rvec, c_idx])
                    rows_out[r, pl.ds(c, _LANES)] = gf + wvec * (gc - gf)

            pltpu.sync_copy(rows_out, o_hbm.at[bi, pl.ds(rbase, rows_per_tile)])

    return _sc_gather_body


def kernel(distance):
    b, l, d = distance.shape
    k = _SC_BATCHES
    bt = b - k
    i32 = jnp.int32

    fl, ce, w = pl.pallas_call(
        _stats_body,
        grid=(k,),
        in_specs=[pl.BlockSpec((1, l, d), lambda i: (i + bt, 0, 0))],
        out_specs=[
            pl.BlockSpec((1, 1, d), lambda i: (i, 0, 0)),
            pl.BlockSpec((1, 1, d), lambda i: (i, 0, 0)),
            pl.BlockSpec((1, 1, d), lambda i: (i, 0, 0)),
        ],
        out_shape=[
            jax.ShapeDtypeStruct((k, 1, d), i32),
            jax.ShapeDtypeStruct((k, 1, d), i32),
            jax.ShapeDtypeStruct((k, 1, d), jnp.float32),
        ],
    )(distance)
    fl, ce, w = fl.reshape(k, d), ce.reshape(k, d), w.reshape(k, d)

    rows_per_tile = l // _TILES
    mesh = plsc.VectorSubcoreMesh(core_axis_name="c", subcore_axis_name="s")
    cp = pltpu.CompilerParams()
    if "needs_layout_passes" in pltpu.CompilerParams.__dataclass_fields__:
        cp = dataclasses.replace(cp, needs_layout_passes=False)
    sc_out = pl.kernel(
        _make_sc_gather_body(bt),
        out_type=jax.ShapeDtypeStruct((k, l, d), jnp.float32),
        mesh=mesh,
        scratch_types=[
            pltpu.VMEM((rows_per_tile, d), jnp.float32),
            pltpu.VMEM((rows_per_tile, d), jnp.float32),
            pltpu.VMEM((d,), i32),
            pltpu.VMEM((d,), i32),
            pltpu.VMEM((d,), jnp.float32),
        ],
        compiler_params=cp,
    )(distance, fl, ce, w)

    tc_out = pl.pallas_call(
        _fused_body,
        grid=(bt,),
        in_specs=[pl.BlockSpec((1, l, d), lambda i: (i, 0, 0))],
        out_specs=pl.BlockSpec((1, l, d), lambda i: (i, 0, 0)),
        out_shape=jax.ShapeDtypeStruct((bt, l, d), distance.dtype),
    )(distance)

    return jnp.concatenate([tc_out, sc_out], axis=0)
